# baseline (speedup 1.0000x reference)
import functools

import jax
import jax.numpy as jnp
from jax import lax
from jax.experimental import pallas as pl
from jax.experimental.pallas import tpu as pltpu
from jax.experimental.pallas import tpu_sc as plsc

NUM_NODES = 10000
N_EDGES = 320000
IN_FEAT = 10000
OUT_FEAT = 128
NUM_RELS = 8
NUM_BASES = 4

NC, NS, L = 2, 16, 16
NW = NC * NS
EPW = N_EDGES // NW
C = 80
NCH = EPW // C
RPT = NUM_NODES // NS
FV = OUT_FEAT // L


_QB = 400
_EB = N_EDGES // C // (IN_FEAT // _QB)


def _embed_body(wc_ref, x0, x1, x2, x3, src_ref, rel_ref, emb_ref, gidx_ref):
    wcs = [[wc_ref[r, b] for b in range(NUM_BASES)] for r in range(NUM_RELS)]
    xrefs = (x0, x1, x2, x3)

    def _step(i, _):
        sl = pl.ds(pl.multiple_of(i * 8, 8), 8)
        xb = [x[sl, 0, 0, :] for x in xrefs]
        for r in range(NUM_RELS):
            acc = wcs[r][0] * xb[0]
            for b in range(1, NUM_BASES):
                acc = acc + wcs[r][b] * xb[b]
            emb_ref[r, sl, :] = acc
        return 0

    lax.fori_loop(0, _QB // 8, _step, 0)
    j = rel_ref[...] * IN_FEAT + src_ref[...]
    gidx_ref[...] = (j & 7) * IN_FEAT + (j >> 3)


def _build_embed(w_comp, w3, src2, rel2):
    w3x = w3.reshape(IN_FEAT, NUM_BASES, 1, OUT_FEAT)

    def _xspec(b):
        return pl.BlockSpec((_QB, 1, 1, OUT_FEAT), lambda i, b=b: (i, b, 0, 0))

    return pl.pallas_call(
        _embed_body,
        grid=(IN_FEAT // _QB,),
        in_specs=[
            pl.BlockSpec(memory_space=pltpu.SMEM),
            _xspec(0), _xspec(1), _xspec(2), _xspec(3),
            pl.BlockSpec((_EB, C), lambda i: (i, 0)),
            pl.BlockSpec((_EB, C), lambda i: (i, 0)),
        ],
        out_specs=[
            pl.BlockSpec((NUM_RELS, _QB, OUT_FEAT), lambda i: (0, i, 0)),
            pl.BlockSpec((_EB, C), lambda i: (i, 0)),
        ],
        out_shape=[
            jax.ShapeDtypeStruct((NUM_RELS, IN_FEAT, OUT_FEAT), jnp.float32),
            jax.ShapeDtypeStruct((N_EDGES // C, C), jnp.int32),
        ],
    )(w_comp, w3x, w3x, w3x, w3x, src2, rel2)


def _sc_body(emb, gidx2, dst2, norm2, out,
             idx_v, dst_v, norm_v, rows0, rows1, h_sh, g0, g1, s0, s1):
    c = lax.axis_index("c")
    s = lax.axis_index("s")
    wid = s * NC + c
    base = wid * NCH
    rows = (rows0, rows1)
    gsem = (g0, g1)
    ssem = (s0, s1)

    pltpu.sync_copy(gidx2.at[pl.ds(base, NCH)], idx_v)
    pltpu.sync_copy(dst2.at[pl.ds(base, NCH)], dst_v)
    pltpu.sync_copy(norm2.at[pl.ds(base, NCH)], norm_v)

    zero = jnp.zeros((L,), jnp.float32)

    def _zb(i, _):
        for k in range(FV):
            rows0[i, pl.ds(k * L, L)] = zero
        return 0

    lax.fori_loop(0, C, _zb, 0)
    for t in range(8):
        off = s * RPT + t * C
        n = C if t < 7 else RPT - 7 * C
        pltpu.sync_copy(rows0.at[pl.ds(0, n)], h_sh.at[pl.ds(off, n)])
    plsc.subcore_barrier()

    def _gather(j, b):
        return pltpu.async_copy(emb.at[idx_v.at[j]], rows[b], gsem[b])

    def _scale(j, b):
        rb = rows[b]

        def _sg(g, _):
            nvec = norm_v[j, pl.ds(g * L, L)]
            for t in range(L):
                nv = nvec[t]
                i = g * L + t
                for k in range(FV):
                    sl = pl.ds(k * L, L)
                    rb[i, sl] = rb[i, sl] * nv
            return 0

        lax.fori_loop(0, C // L, _sg, 0)

    def _scatter(j, b):
        return pltpu.async_copy(rows[b], h_sh.at[dst_v.at[j]], ssem[b], add=True)

    def _drain_g(b):
        pltpu.make_async_copy(emb.at[idx_v.at[0]], rows[b], gsem[b]).wait()

    def _drain_s(b):
        pltpu.make_async_copy(rows[b], h_sh.at[dst_v.at[0]], ssem[b]).wait()

    _gather(0, 0)
    _gather(1, 1)

    def _turn(j, b):
        _drain_g(b)
        _scale(j, b)
        _scatter(j, b)
        _drain_s(b)
        _gather(j + 2, b)

    def _main(jj, _):
        _turn(2 * jj, 0)
        _turn(2 * jj + 1, 1)
        return 0

    lax.fori_loop(0, (NCH - 3) // 2, _main, 0)
    _drain_g(0)
    _scale(NCH - 3, 0)
    _scatter(NCH - 3, 0)
    _drain_s(0)
    _gather(NCH - 1, 0)
    _drain_g(1)
    _scale(NCH - 2, 1)
    _scatter(NCH - 2, 1)
    _drain_s(1)
    _drain_g(0)
    _scale(NCH - 1, 0)
    _scatter(NCH - 1, 0)
    _drain_s(0)
    plsc.subcore_barrier()

    for t in range(8):
        off = s * RPT + t * C
        n = C if t < 7 else RPT - 7 * C
        pltpu.sync_copy(h_sh.at[pl.ds(off, n)], rows0.at[pl.ds(0, n)])
        pltpu.sync_copy(rows0.at[pl.ds(0, n)], out.at[c, pl.ds(off, n)])


def _sc_call(emb, gidx2, dst2, norm2):
    mesh = plsc.VectorSubcoreMesh(core_axis_name="c", subcore_axis_name="s")
    f = pl.kernel(
        _sc_body,
        out_type=jax.ShapeDtypeStruct((NC, NUM_NODES, OUT_FEAT), jnp.float32),
        mesh=mesh,
        compiler_params=pltpu.CompilerParams(use_tc_tiling_on_sc=False),
        scratch_types=[
            pltpu.VMEM((NCH, C), jnp.int32),
            pltpu.VMEM((NCH, C), jnp.int32),
            pltpu.VMEM((NCH, C), jnp.float32),
            pltpu.VMEM((C, OUT_FEAT), jnp.float32),
            pltpu.VMEM((C, OUT_FEAT), jnp.float32),
            pltpu.VMEM_SHARED((NUM_NODES, OUT_FEAT), jnp.float32),
            pltpu.SemaphoreType.DMA,
            pltpu.SemaphoreType.DMA,
            pltpu.SemaphoreType.DMA,
            pltpu.SemaphoreType.DMA,
        ],
    )
    return f(emb, gidx2, dst2, norm2)


_RB = 2000


def _sum_body(p_ref, o_ref):
    o_ref[...] = p_ref[0] + p_ref[1]


def _sum_partials(partial):
    return pl.pallas_call(
        _sum_body,
        grid=(NUM_NODES // _RB,),
        in_specs=[pl.BlockSpec((NC, _RB, OUT_FEAT), lambda i: (0, i, 0))],
        out_specs=pl.BlockSpec((_RB, OUT_FEAT), lambda i: (i, 0)),
        out_shape=jax.ShapeDtypeStruct((NUM_NODES, OUT_FEAT), jnp.float32),
    )(partial)


def kernel(src_id, dst_id, rel_type, norm, weight, w_comp):
    w3 = weight.reshape(IN_FEAT, NUM_BASES, OUT_FEAT)
    embT, gidx2 = _build_embed(
        w_comp, w3, src_id.reshape(-1, C), rel_type.reshape(-1, C))
    emb = embT.reshape(NUM_RELS * IN_FEAT, OUT_FEAT)
    dst2 = dst_id.reshape(-1, C)
    norm2 = norm.reshape(-1, C)
    partial = _sc_call(emb, gidx2, dst2, norm2)
    return _sum_partials(partial)


# speedup vs baseline: 1.0969x; 1.0969x over previous
import functools

import jax
import jax.numpy as jnp
from jax import lax
from jax.experimental import pallas as pl
from jax.experimental.pallas import tpu as pltpu
from jax.experimental.pallas import tpu_sc as plsc

NUM_NODES = 10000
N_EDGES = 320000
IN_FEAT = 10000
OUT_FEAT = 128
NUM_RELS = 8
NUM_BASES = 4

NC, NS, L = 2, 16, 16
NW = NC * NS
EPW = N_EDGES // NW
C = 80
NCH = EPW // C
RPT = NUM_NODES // NS
FV = OUT_FEAT // L


_QB = 400
_G = 16
_EB = N_EDGES // C // (IN_FEAT // _QB)


def _embed_body(a_ref, w2_ref, src_ref, rel_ref, emb_ref, gidx_ref):
    a = a_ref[...]

    def _step(g, _):
        x = w2_ref[pl.ds(g * 4 * _G, 4 * _G), :]
        y = lax.dot_general(a, x, (((1,), (0,)), ((), ())),
                            preferred_element_type=jnp.float32)
        emb_ref[pl.ds(g * 8 * _G, 8 * _G), :] = y
        return 0

    lax.fori_loop(0, _QB // _G, _step, 0)
    gidx_ref[...] = rel_ref[...] * IN_FEAT + src_ref[...]


def _build_embed(w_comp, w2, src2, rel2):
    a = jnp.kron(jnp.eye(_G, dtype=jnp.float32), w_comp)

    return pl.pallas_call(
        _embed_body,
        grid=(IN_FEAT // _QB,),
        in_specs=[
            pl.BlockSpec((8 * _G, 4 * _G), lambda i: (0, 0)),
            pl.BlockSpec((NUM_BASES * _QB, OUT_FEAT), lambda i: (i, 0)),
            pl.BlockSpec((_EB, C), lambda i: (i, 0)),
            pl.BlockSpec((_EB, C), lambda i: (i, 0)),
        ],
        out_specs=[
            pl.BlockSpec((NUM_RELS * _QB, OUT_FEAT), lambda i: (i, 0)),
            pl.BlockSpec((_EB, C), lambda i: (i, 0)),
        ],
        out_shape=[
            jax.ShapeDtypeStruct((NUM_RELS * IN_FEAT, OUT_FEAT), jnp.float32),
            jax.ShapeDtypeStruct((N_EDGES // C, C), jnp.int32),
        ],
    )(a, w2, src2, rel2)


def _sc_body(emb, gidx2, dst2, norm2, out,
             idx_v, dst_v, norm_v, rows0, rows1, h_sh, g0, g1, s0, s1):
    c = lax.axis_index("c")
    s = lax.axis_index("s")
    wid = s * NC + c
    base = wid * NCH
    rows = (rows0, rows1)
    gsem = (g0, g1)
    ssem = (s0, s1)

    pltpu.sync_copy(gidx2.at[pl.ds(base, NCH)], idx_v)
    pltpu.sync_copy(dst2.at[pl.ds(base, NCH)], dst_v)
    pltpu.sync_copy(norm2.at[pl.ds(base, NCH)], norm_v)

    zero = jnp.zeros((L,), jnp.float32)

    def _zb(i, _):
        for k in range(FV):
            rows0[i, pl.ds(k * L, L)] = zero
        return 0

    lax.fori_loop(0, C, _zb, 0)
    for t in range(8):
        off = s * RPT + t * C
        n = C if t < 7 else RPT - 7 * C
        pltpu.sync_copy(rows0.at[pl.ds(0, n)], h_sh.at[pl.ds(off, n)])
    plsc.subcore_barrier()

    def _gather(j, b):
        return pltpu.async_copy(emb.at[idx_v.at[j]], rows[b], gsem[b])

    def _scale(j, b):
        rb = rows[b]

        def _sg(g, _):
            nvec = norm_v[j, pl.ds(g * L, L)]
            for t in range(L):
                nv = nvec[t]
                i = g * L + t
                for k in range(FV):
                    sl = pl.ds(k * L, L)
                    rb[i, sl] = rb[i, sl] * nv
            return 0

        lax.fori_loop(0, C // L, _sg, 0)

    def _scatter(j, b):
        return pltpu.async_copy(rows[b], h_sh.at[dst_v.at[j]], ssem[b], add=True)

    def _drain_g(b):
        pltpu.make_async_copy(emb.at[idx_v.at[0]], rows[b], gsem[b]).wait()

    def _drain_s(b):
        pltpu.make_async_copy(rows[b], h_sh.at[dst_v.at[0]], ssem[b]).wait()

    _gather(0, 0)
    _gather(1, 1)

    def _turn(j, b):
        _drain_g(b)
        _scale(j, b)
        _scatter(j, b)
        _drain_s(b)
        _gather(j + 2, b)

    def _main(jj, _):
        _turn(2 * jj, 0)
        _turn(2 * jj + 1, 1)
        return 0

    lax.fori_loop(0, (NCH - 3) // 2, _main, 0)
    _drain_g(0)
    _scale(NCH - 3, 0)
    _scatter(NCH - 3, 0)
    _drain_s(0)
    _gather(NCH - 1, 0)
    _drain_g(1)
    _scale(NCH - 2, 1)
    _scatter(NCH - 2, 1)
    _drain_s(1)
    _drain_g(0)
    _scale(NCH - 1, 0)
    _scatter(NCH - 1, 0)
    _drain_s(0)
    plsc.subcore_barrier()

    for t in range(8):
        off = s * RPT + t * C
        n = C if t < 7 else RPT - 7 * C
        pltpu.sync_copy(h_sh.at[pl.ds(off, n)], rows0.at[pl.ds(0, n)])
        pltpu.sync_copy(rows0.at[pl.ds(0, n)], out.at[c, pl.ds(off, n)])


def _sc_call(emb, gidx2, dst2, norm2):
    mesh = plsc.VectorSubcoreMesh(core_axis_name="c", subcore_axis_name="s")
    f = pl.kernel(
        _sc_body,
        out_type=jax.ShapeDtypeStruct((NC, NUM_NODES, OUT_FEAT), jnp.float32),
        mesh=mesh,
        compiler_params=pltpu.CompilerParams(use_tc_tiling_on_sc=False),
        scratch_types=[
            pltpu.VMEM((NCH, C), jnp.int32),
            pltpu.VMEM((NCH, C), jnp.int32),
            pltpu.VMEM((NCH, C), jnp.float32),
            pltpu.VMEM((C, OUT_FEAT), jnp.float32),
            pltpu.VMEM((C, OUT_FEAT), jnp.float32),
            pltpu.VMEM_SHARED((NUM_NODES, OUT_FEAT), jnp.float32),
            pltpu.SemaphoreType.DMA,
            pltpu.SemaphoreType.DMA,
            pltpu.SemaphoreType.DMA,
            pltpu.SemaphoreType.DMA,
        ],
    )
    return f(emb, gidx2, dst2, norm2)


_RB = 2000


def _sum_body(p_ref, o_ref):
    o_ref[...] = p_ref[0] + p_ref[1]


def _sum_partials(partial):
    return pl.pallas_call(
        _sum_body,
        grid=(NUM_NODES // _RB,),
        in_specs=[pl.BlockSpec((NC, _RB, OUT_FEAT), lambda i: (0, i, 0))],
        out_specs=pl.BlockSpec((_RB, OUT_FEAT), lambda i: (i, 0)),
        out_shape=jax.ShapeDtypeStruct((NUM_NODES, OUT_FEAT), jnp.float32),
    )(partial)


def kernel(src_id, dst_id, rel_type, norm, weight, w_comp):
    w2 = weight.reshape(NUM_BASES * IN_FEAT, OUT_FEAT)
    emb, gidx2 = _build_embed(
        w_comp, w2, src_id.reshape(-1, C), rel_type.reshape(-1, C))
    dst2 = dst_id.reshape(-1, C)
    norm2 = norm.reshape(-1, C)
    partial = _sc_call(emb, gidx2, dst2, norm2)
    return _sum_partials(partial)


# speedup vs baseline: 1.3526x; 1.2331x over previous
import functools

import jax
import jax.numpy as jnp
from jax import lax
from jax.experimental import pallas as pl
from jax.experimental.pallas import tpu as pltpu
from jax.experimental.pallas import tpu_sc as plsc

NUM_NODES = 10000
N_EDGES = 320000
IN_FEAT = 10000
OUT_FEAT = 128
NUM_RELS = 8
NUM_BASES = 4

NC, NS, L = 2, 16, 16
NW = NC * NS
EPW = N_EDGES // NW
C = 80
NCH = EPW // C
RPT = NUM_NODES // NS
FV = OUT_FEAT // L


_QB = 400
_G = 16
_EB = N_EDGES // C // (IN_FEAT // _QB)


def _embed_body(a_ref, w2_ref, src_ref, rel_ref, emb_ref, gidx_ref):
    a = a_ref[...]

    for g in range(_QB // _G):
        x = w2_ref[pl.ds(g * 4 * _G, 4 * _G), :]
        y = lax.dot_general(a, x, (((1,), (0,)), ((), ())),
                            preferred_element_type=jnp.float32)
        emb_ref[pl.ds(g * 8 * _G, 8 * _G), :] = y
    gidx_ref[...] = rel_ref[...] * IN_FEAT + src_ref[...]


def _build_embed(w_comp, w2, src2, rel2):
    a = jnp.kron(jnp.eye(_G, dtype=jnp.float32), w_comp)

    return pl.pallas_call(
        _embed_body,
        grid=(IN_FEAT // _QB,),
        in_specs=[
            pl.BlockSpec((8 * _G, 4 * _G), lambda i: (0, 0)),
            pl.BlockSpec((NUM_BASES * _QB, OUT_FEAT), lambda i: (i, 0)),
            pl.BlockSpec((_EB, C), lambda i: (i, 0)),
            pl.BlockSpec((_EB, C), lambda i: (i, 0)),
        ],
        out_specs=[
            pl.BlockSpec((NUM_RELS * _QB, OUT_FEAT), lambda i: (i, 0)),
            pl.BlockSpec((_EB, C), lambda i: (i, 0)),
        ],
        out_shape=[
            jax.ShapeDtypeStruct((NUM_RELS * IN_FEAT, OUT_FEAT), jnp.float32),
            jax.ShapeDtypeStruct((N_EDGES // C, C), jnp.int32),
        ],
    )(a, w2, src2, rel2)


def _sc_body(emb, gidx2, dst2, norm2, out,
             idx_v, dst_v, norm_v, rows0, rows1, h_sh, g0, g1, s0, s1):
    c = lax.axis_index("c")
    s = lax.axis_index("s")
    wid = s * NC + c
    base = wid * NCH
    rows = (rows0, rows1)
    gsem = (g0, g1)
    ssem = (s0, s1)

    pltpu.sync_copy(gidx2.at[pl.ds(base, NCH)], idx_v)
    pltpu.sync_copy(dst2.at[pl.ds(base, NCH)], dst_v)
    pltpu.sync_copy(norm2.at[pl.ds(base, NCH)], norm_v)

    zero = jnp.zeros((L,), jnp.float32)

    def _zb(i, _):
        for k in range(FV):
            rows0[i, pl.ds(k * L, L)] = zero
        return 0

    lax.fori_loop(0, C, _zb, 0)
    for t in range(8):
        off = s * RPT + t * C
        n = C if t < 7 else RPT - 7 * C
        pltpu.sync_copy(rows0.at[pl.ds(0, n)], h_sh.at[pl.ds(off, n)])
    plsc.subcore_barrier()

    def _gather(j, b):
        return pltpu.async_copy(emb.at[idx_v.at[j]], rows[b], gsem[b])

    def _scale(j, b):
        rb = rows[b]

        def _sg(g, _):
            nvec = norm_v[j, pl.ds(g * L, L)]
            for t in range(L):
                nv = nvec[t]
                i = g * L + t
                for k in range(FV):
                    sl = pl.ds(k * L, L)
                    rb[i, sl] = rb[i, sl] * nv
            return 0

        lax.fori_loop(0, C // L, _sg, 0)

    def _scatter(j, b):
        return pltpu.async_copy(rows[b], h_sh.at[dst_v.at[j]], ssem[b], add=True)

    def _drain_g(b):
        pltpu.make_async_copy(emb.at[idx_v.at[0]], rows[b], gsem[b]).wait()

    def _drain_s(b):
        pltpu.make_async_copy(rows[b], h_sh.at[dst_v.at[0]], ssem[b]).wait()

    _gather(0, 0)
    _gather(1, 1)

    def _turn(j, b):
        _drain_g(b)
        _scale(j, b)
        _scatter(j, b)
        _drain_s(b)
        _gather(j + 2, b)

    def _main(jj, _):
        _turn(2 * jj, 0)
        _turn(2 * jj + 1, 1)
        return 0

    lax.fori_loop(0, (NCH - 3) // 2, _main, 0)
    _drain_g(0)
    _scale(NCH - 3, 0)
    _scatter(NCH - 3, 0)
    _drain_s(0)
    _gather(NCH - 1, 0)
    _drain_g(1)
    _scale(NCH - 2, 1)
    _scatter(NCH - 2, 1)
    _drain_s(1)
    _drain_g(0)
    _scale(NCH - 1, 0)
    _scatter(NCH - 1, 0)
    _drain_s(0)
    plsc.subcore_barrier()

    for t in range(8):
        off = s * RPT + t * C
        n = C if t < 7 else RPT - 7 * C
        pltpu.sync_copy(h_sh.at[pl.ds(off, n)], rows0.at[pl.ds(0, n)])
        pltpu.sync_copy(rows0.at[pl.ds(0, n)], out.at[c, pl.ds(off, n)])


def _sc_call(emb, gidx2, dst2, norm2):
    mesh = plsc.VectorSubcoreMesh(core_axis_name="c", subcore_axis_name="s")
    f = pl.kernel(
        _sc_body,
        out_type=jax.ShapeDtypeStruct((NC, NUM_NODES, OUT_FEAT), jnp.float32),
        mesh=mesh,
        compiler_params=pltpu.CompilerParams(use_tc_tiling_on_sc=False),
        scratch_types=[
            pltpu.VMEM((NCH, C), jnp.int32),
            pltpu.VMEM((NCH, C), jnp.int32),
            pltpu.VMEM((NCH, C), jnp.float32),
            pltpu.VMEM((C, OUT_FEAT), jnp.float32),
            pltpu.VMEM((C, OUT_FEAT), jnp.float32),
            pltpu.VMEM_SHARED((NUM_NODES, OUT_FEAT), jnp.float32),
            pltpu.SemaphoreType.DMA,
            pltpu.SemaphoreType.DMA,
            pltpu.SemaphoreType.DMA,
            pltpu.SemaphoreType.DMA,
        ],
    )
    return f(emb, gidx2, dst2, norm2)


_RB = 2000


def _sum_body(p_ref, o_ref):
    o_ref[...] = p_ref[0] + p_ref[1]


def _sum_partials(partial):
    return pl.pallas_call(
        _sum_body,
        grid=(NUM_NODES // _RB,),
        in_specs=[pl.BlockSpec((NC, _RB, OUT_FEAT), lambda i: (0, i, 0))],
        out_specs=pl.BlockSpec((_RB, OUT_FEAT), lambda i: (i, 0)),
        out_shape=jax.ShapeDtypeStruct((NUM_NODES, OUT_FEAT), jnp.float32),
    )(partial)


def kernel(src_id, dst_id, rel_type, norm, weight, w_comp):
    w2 = weight.reshape(NUM_BASES * IN_FEAT, OUT_FEAT)
    emb, gidx2 = _build_embed(
        w_comp, w2, src_id.reshape(-1, C), rel_type.reshape(-1, C))
    dst2 = dst_id.reshape(-1, C)
    norm2 = norm.reshape(-1, C)
    partial = _sc_call(emb, gidx2, dst2, norm2)
    return _sum_partials(partial)


# speedup vs baseline: 1.4549x; 1.0756x over previous
import functools

import jax
import jax.numpy as jnp
from jax import lax
from jax.experimental import pallas as pl
from jax.experimental.pallas import tpu as pltpu
from jax.experimental.pallas import tpu_sc as plsc

NUM_NODES = 10000
N_EDGES = 320000
IN_FEAT = 10000
OUT_FEAT = 128
NUM_RELS = 8
NUM_BASES = 4

NC, NS, L = 2, 16, 16
NW = NC * NS
EPW = N_EDGES // NW
C = 80
NCH = EPW // C
RPT = NUM_NODES // NS
FV = OUT_FEAT // L


_QB = 400
_G = 16
_EB = N_EDGES // C // (IN_FEAT // _QB)


def _embed_body(a_ref, w2_ref, src_ref, rel_ref, dst_ref, emb_ref, pix_ref):
    a = a_ref[...]

    for g in range(_QB // _G):
        x = w2_ref[pl.ds(g * 4 * _G, 4 * _G), :]
        y = lax.dot_general(a, x, (((1,), (0,)), ((), ())),
                            preferred_element_type=jnp.float32)
        emb_ref[pl.ds(g * 8 * _G, 8 * _G), :] = y
    j = rel_ref[...] * IN_FEAT + src_ref[...]
    pix_ref[...] = (j << 14) + dst_ref[...]


def _build_embed(w_comp, w2, src2, rel2, dst2):
    a = jnp.kron(jnp.eye(_G, dtype=jnp.float32), w_comp)
    espec = pl.BlockSpec((_EB, C), lambda i: (i, 0))

    return pl.pallas_call(
        _embed_body,
        grid=(IN_FEAT // _QB,),
        in_specs=[
            pl.BlockSpec((8 * _G, 4 * _G), lambda i: (0, 0)),
            pl.BlockSpec((NUM_BASES * _QB, OUT_FEAT), lambda i: (i, 0)),
            espec, espec, espec,
        ],
        out_specs=[
            pl.BlockSpec((NUM_RELS * _QB, OUT_FEAT), lambda i: (i, 0)),
            espec,
        ],
        out_shape=[
            jax.ShapeDtypeStruct((NUM_RELS * IN_FEAT, OUT_FEAT), jnp.float32),
            jax.ShapeDtypeStruct((N_EDGES // C, C), jnp.int32),
        ],
    )(a, w2, src2, rel2, dst2)


NA = 63
NB = NCH - NA


def _sc_body(emb, pix2, norm2, out,
             pix_v, norm_v, idxc, dstc, rows0, rows1, rows2,
             h_sh, g0, g1, g2, s0, s1, s2):
    c = lax.axis_index("c")
    s = lax.axis_index("s")
    wid = s * NC + c
    base = wid * NCH
    rows = (rows0, rows1, rows2)
    gsem = (g0, g1, g2)
    ssem = (s0, s1, s2)

    pltpu.sync_copy(pix2.at[pl.ds(base, NCH)], pix_v)
    pltpu.sync_copy(norm2.at[pl.ds(base, NA)], norm_v)

    zero = jnp.zeros((L,), jnp.float32)

    def _zb(i, _):
        for k in range(FV):
            rows0[i, pl.ds(k * L, L)] = zero
        return 0

    lax.fori_loop(0, C, _zb, 0)
    for t in range(8):
        off = s * RPT + t * C
        n = C if t < 7 else RPT - 7 * C
        pltpu.sync_copy(rows0.at[pl.ds(0, n)], h_sh.at[pl.ds(off, n)])
    plsc.subcore_barrier()

    def _unpack(j, b):
        for k in range(C // L):
            sl = pl.ds(k * L, L)
            p = pix_v[j, sl]
            idxc[b, sl] = p >> 14
            dstc[b, sl] = p & 16383

    def _gather(b):
        pltpu.async_copy(emb.at[idxc.at[b]], rows[b], gsem[b])

    def _scale(jn, b):
        rb = rows[b]

        def _sg(g, _):
            nvec = norm_v[jn, pl.ds(g * L, L)]
            for t in range(L):
                nv = nvec[t]
                i = g * L + t
                for k in range(FV):
                    sl = pl.ds(k * L, L)
                    rb[i, sl] = rb[i, sl] * nv
            return 0

        lax.fori_loop(0, C // L, _sg, 0)

    def _scatter(b):
        pltpu.async_copy(rows[b], h_sh.at[dstc.at[b]], ssem[b], add=True)

    def _drain_g(b):
        pltpu.make_async_copy(emb.at[idxc.at[b]], rows[b], gsem[b]).wait()

    def _drain_s(b):
        pltpu.make_async_copy(rows[b], h_sh.at[dstc.at[b]], ssem[b]).wait()

    _unpack(0, 0)
    _gather(0)
    _unpack(1, 1)
    _gather(1)

    def _turn(j, t, o):
        b = t
        b2 = (t + 2) % 3
        _drain_g(b)
        _scale(j - o, b)
        _scatter(b)

        @pl.when(j >= 1)
        def _():
            _drain_s(b2)

        @pl.when(j < NCH - 2)
        def _():
            _unpack(j + 2, b2)
            _gather(b2)

    def _mainA(jj, _):
        for t in range(3):
            _turn(3 * jj + t, t, 0)
        return 0

    lax.fori_loop(0, NA // 3, _mainA, 0)

    pltpu.sync_copy(norm2.at[pl.ds(base + NA, NB)], norm_v.at[pl.ds(0, NB)])

    def _mainB(jj, _):
        for t in range(3):
            _turn(NA + 3 * jj + t, t, NA)
        return 0

    lax.fori_loop(0, (NB - 2) // 3, _mainB, 0)
    _turn(NCH - 2, 0, NA)
    _turn(NCH - 1, 1, NA)
    _drain_s(1)
    plsc.subcore_barrier()

    for t in range(8):
        off = s * RPT + t * C
        n = C if t < 7 else RPT - 7 * C
        pltpu.sync_copy(h_sh.at[pl.ds(off, n)], rows0.at[pl.ds(0, n)])
        pltpu.sync_copy(rows0.at[pl.ds(0, n)], out.at[c, pl.ds(off, n)])


def _sc_call(emb, pix2, norm2):
    mesh = plsc.VectorSubcoreMesh(core_axis_name="c", subcore_axis_name="s")
    f = pl.kernel(
        _sc_body,
        out_type=jax.ShapeDtypeStruct((NC, NUM_NODES, OUT_FEAT), jnp.float32),
        mesh=mesh,
        compiler_params=pltpu.CompilerParams(use_tc_tiling_on_sc=False),
        scratch_types=[
            pltpu.VMEM((NCH, C), jnp.int32),
            pltpu.VMEM((NA, C), jnp.float32),
            pltpu.VMEM((3, C), jnp.int32),
            pltpu.VMEM((3, C), jnp.int32),
            pltpu.VMEM((C, OUT_FEAT), jnp.float32),
            pltpu.VMEM((C, OUT_FEAT), jnp.float32),
            pltpu.VMEM((C, OUT_FEAT), jnp.float32),
            pltpu.VMEM_SHARED((NUM_NODES, OUT_FEAT), jnp.float32),
            pltpu.SemaphoreType.DMA, pltpu.SemaphoreType.DMA, pltpu.SemaphoreType.DMA,
            pltpu.SemaphoreType.DMA, pltpu.SemaphoreType.DMA, pltpu.SemaphoreType.DMA,
        ],
    )
    return f(emb, pix2, norm2)


_RB = 2000


def _sum_body(p_ref, o_ref):
    o_ref[...] = p_ref[0] + p_ref[1]


def _sum_partials(partial):
    return pl.pallas_call(
        _sum_body,
        grid=(NUM_NODES // _RB,),
        in_specs=[pl.BlockSpec((NC, _RB, OUT_FEAT), lambda i: (0, i, 0))],
        out_specs=pl.BlockSpec((_RB, OUT_FEAT), lambda i: (i, 0)),
        out_shape=jax.ShapeDtypeStruct((NUM_NODES, OUT_FEAT), jnp.float32),
    )(partial)


def kernel(src_id, dst_id, rel_type, norm, weight, w_comp):
    w2 = weight.reshape(NUM_BASES * IN_FEAT, OUT_FEAT)
    emb, pix2 = _build_embed(
        w_comp, w2, src_id.reshape(-1, C), rel_type.reshape(-1, C),
        dst_id.reshape(-1, C))
    norm2 = norm.reshape(-1, C)
    partial = _sc_call(emb, pix2, norm2)
    return _sum_partials(partial)


# speedup vs baseline: 1.4603x; 1.0037x over previous
import functools

import jax
import jax.numpy as jnp
from jax import lax
from jax.experimental import pallas as pl
from jax.experimental.pallas import tpu as pltpu
from jax.experimental.pallas import tpu_sc as plsc

NUM_NODES = 10000
N_EDGES = 320000
IN_FEAT = 10000
OUT_FEAT = 128
NUM_RELS = 8
NUM_BASES = 4

NC, NS, L = 2, 16, 16
NW = NC * NS
EPW = N_EDGES // NW
C = 80
NCH = EPW // C
RPT = NUM_NODES // NS
FV = OUT_FEAT // L


_QB = 400
_G = 16
_EB = N_EDGES // C // (IN_FEAT // _QB)


def _embed_body(a_ref, w2_ref, src_ref, rel_ref, dst_ref, emb_ref, pix_ref):
    a = a_ref[...]

    for g in range(_QB // _G):
        x = w2_ref[pl.ds(g * 4 * _G, 4 * _G), :]
        y = lax.dot_general(a, x, (((1,), (0,)), ((), ())),
                            preferred_element_type=jnp.float32)
        emb_ref[pl.ds(g * 8 * _G, 8 * _G), :] = y
    j = rel_ref[...] * IN_FEAT + src_ref[...]
    pix_ref[...] = (j << 14) + dst_ref[...]


def _build_embed(w_comp, w2, src2, rel2, dst2):
    a = jnp.kron(jnp.eye(_G, dtype=jnp.float32), w_comp)
    espec = pl.BlockSpec((_EB, C), lambda i: (i, 0))

    return pl.pallas_call(
        _embed_body,
        grid=(IN_FEAT // _QB,),
        in_specs=[
            pl.BlockSpec((8 * _G, 4 * _G), lambda i: (0, 0)),
            pl.BlockSpec((NUM_BASES * _QB, OUT_FEAT), lambda i: (i, 0)),
            espec, espec, espec,
        ],
        out_specs=[
            pl.BlockSpec((NUM_RELS * _QB, OUT_FEAT), lambda i: (i, 0)),
            espec,
        ],
        out_shape=[
            jax.ShapeDtypeStruct((NUM_RELS * IN_FEAT, OUT_FEAT), jnp.float32),
            jax.ShapeDtypeStruct((N_EDGES // C, C), jnp.int32),
        ],
    )(a, w2, src2, rel2, dst2)


NA = 63
NB = NCH - NA


def _sc_body(emb, pix2, norm2, out,
             pix_v, norm_v, idxc, dstc, rows0, rows1, rows2,
             h_sh, g0, g1, g2, s0, s1, s2):
    c = lax.axis_index("c")
    s = lax.axis_index("s")
    wid = s * NC + c
    base = wid * NCH
    rows = (rows0, rows1, rows2)
    gsem = (g0, g1, g2)
    ssem = (s0, s1, s2)

    pltpu.sync_copy(pix2.at[pl.ds(base, NCH)], pix_v)
    pltpu.sync_copy(norm2.at[pl.ds(base, NA)], norm_v)

    zero = jnp.zeros((L,), jnp.float32)

    def _zb(i, _):
        for k in range(FV):
            rows0[i, pl.ds(k * L, L)] = zero
        return 0

    lax.fori_loop(0, C, _zb, 0)
    for t in range(8):
        off = s * RPT + t * C
        n = C if t < 7 else RPT - 7 * C
        pltpu.sync_copy(rows0.at[pl.ds(0, n)], h_sh.at[pl.ds(off, n)])
    plsc.subcore_barrier()

    def _unpack(j, b):
        for k in range(C // L):
            sl = pl.ds(k * L, L)
            p = pix_v[j, sl]
            idxc[b, sl] = p >> 14
            dstc[b, sl] = p & 16383

    def _gather(b):
        pltpu.async_copy(emb.at[idxc.at[b]], rows[b], gsem[b])

    def _scale(jn, b):
        rb = rows[b]

        def _sg(g, _):
            nvec = norm_v[jn, pl.ds(g * L, L)]
            for t in range(L):
                nv = nvec[t]
                i = g * L + t
                for k in range(FV):
                    sl = pl.ds(k * L, L)
                    rb[i, sl] = rb[i, sl] * nv
            return 0

        lax.fori_loop(0, C // L, _sg, 0)

    def _scatter(b):
        pltpu.async_copy(rows[b], h_sh.at[dstc.at[b]], ssem[b], add=True)

    def _drain_g(b):
        pltpu.make_async_copy(emb.at[idxc.at[b]], rows[b], gsem[b]).wait()

    def _drain_s(b):
        pltpu.make_async_copy(rows[b], h_sh.at[dstc.at[b]], ssem[b]).wait()

    _unpack(0, 0)
    _gather(0)
    _unpack(1, 1)
    _gather(1)

    def _turn(j, t, o):
        b = t
        b2 = (t + 2) % 3
        _drain_g(b)
        _scale(j - o, b)
        _scatter(b)

        @pl.when(j >= 1)
        def _():
            _drain_s(b2)

        @pl.when(j < NCH - 2)
        def _():
            _unpack(j + 2, b2)
            _gather(b2)

    def _mainA(jj, _):
        for t in range(3):
            _turn(3 * jj + t, t, 0)
        return 0

    lax.fori_loop(0, NA // 3, _mainA, 0)

    pltpu.sync_copy(norm2.at[pl.ds(base + NA, NB)], norm_v.at[pl.ds(0, NB)])

    def _mainB(jj, _):
        for t in range(3):
            _turn(NA + 3 * jj + t, t, NA)
        return 0

    lax.fori_loop(0, (NB - 2) // 3, _mainB, 0)
    _turn(NCH - 2, 0, NA)
    _turn(NCH - 1, 1, NA)
    _drain_s(1)
    plsc.subcore_barrier()

    sl = pl.ds(s * RPT, RPT)
    pltpu.sync_copy(h_sh.at[sl], out.at[c, sl])


def _sc_call(emb, pix2, norm2):
    mesh = plsc.VectorSubcoreMesh(core_axis_name="c", subcore_axis_name="s")
    f = pl.kernel(
        _sc_body,
        out_type=jax.ShapeDtypeStruct((NC, NUM_NODES, OUT_FEAT), jnp.float32),
        mesh=mesh,
        compiler_params=pltpu.CompilerParams(use_tc_tiling_on_sc=False),
        scratch_types=[
            pltpu.VMEM((NCH, C), jnp.int32),
            pltpu.VMEM((NA, C), jnp.float32),
            pltpu.VMEM((3, C), jnp.int32),
            pltpu.VMEM((3, C), jnp.int32),
            pltpu.VMEM((C, OUT_FEAT), jnp.float32),
            pltpu.VMEM((C, OUT_FEAT), jnp.float32),
            pltpu.VMEM((C, OUT_FEAT), jnp.float32),
            pltpu.VMEM_SHARED((NUM_NODES, OUT_FEAT), jnp.float32),
            pltpu.SemaphoreType.DMA, pltpu.SemaphoreType.DMA, pltpu.SemaphoreType.DMA,
            pltpu.SemaphoreType.DMA, pltpu.SemaphoreType.DMA, pltpu.SemaphoreType.DMA,
        ],
    )
    return f(emb, pix2, norm2)


_RB = 2000


def _sum_body(p_ref, o_ref):
    o_ref[...] = p_ref[0] + p_ref[1]


def _sum_partials(partial):
    return pl.pallas_call(
        _sum_body,
        grid=(NUM_NODES // _RB,),
        in_specs=[pl.BlockSpec((NC, _RB, OUT_FEAT), lambda i: (0, i, 0))],
        out_specs=pl.BlockSpec((_RB, OUT_FEAT), lambda i: (i, 0)),
        out_shape=jax.ShapeDtypeStruct((NUM_NODES, OUT_FEAT), jnp.float32),
    )(partial)


def kernel(src_id, dst_id, rel_type, norm, weight, w_comp):
    w2 = weight.reshape(NUM_BASES * IN_FEAT, OUT_FEAT)
    emb, pix2 = _build_embed(
        w_comp, w2, src_id.reshape(-1, C), rel_type.reshape(-1, C),
        dst_id.reshape(-1, C))
    norm2 = norm.reshape(-1, C)
    partial = _sc_call(emb, pix2, norm2)
    return _sum_partials(partial)
